# Initial kernel scaffold; baseline (speedup 1.0000x reference)
import jax
import jax.numpy as jnp
from jax.experimental import pallas as pl


def kernel(x, emb_table, pos_table):
    raise NotImplementedError("write your pallas kernel here")


# speedup vs baseline: 5.4909x; 5.4909x over previous
import functools

import jax
import jax.numpy as jnp
import numpy as np
from jax import lax
from jax.experimental import pallas as pl
from jax.experimental.pallas import tpu as pltpu
from jax.experimental.pallas import tpu_sc as plsc

B, P, V, D = 16384, 12, 28, 24
N = B * P
NC, NS, L = 2, 16, 16
NW = NC * NS
ROWS_W = N // NW
SG_ROWS = 3 * L
SG_PER_CHUNK = 16
CHUNK_ROWS = SG_ROWS * SG_PER_CHUNK
CHUNK_ELEMS = CHUNK_ROWS * D
NCHUNKS = ROWS_W // CHUNK_ROWS

_GDN = lax.GatherDimensionNumbers(
    offset_dims=(), collapsed_slice_dims=(0,), start_index_map=(0,))


def _dyn_gather(vec, idx):
    return lax.gather(vec, idx[:, None], _GDN, slice_sizes=(1,),
                      mode=lax.GatherScatterMode.PROMISE_IN_BOUNDS)


_mesh = plsc.VectorSubcoreMesh(core_axis_name="c", subcore_axis_name="s")


@functools.partial(
    pl.kernel,
    mesh=_mesh,
    out_type=jax.ShapeDtypeStruct((N * D,), jnp.float32),
    compiler_params=pltpu.CompilerParams(needs_layout_passes=False),
    scratch_types=[
        pltpu.VMEM((ROWS_W,), jnp.int32),
        pltpu.VMEM((V * D,), jnp.float32),
        pltpu.VMEM((P * D,), jnp.float32),
        pltpu.VMEM((CHUNK_ELEMS,), jnp.float32),
    ],
)
def _lookup(x_hbm, emb_hbm, pos_hbm, out_hbm, x_v, emb_v, pos_v, buf_v):
    wid = lax.axis_index("s") * NC + lax.axis_index("c")
    base = wid * ROWS_W

    pltpu.sync_copy(x_hbm.at[pl.ds(base, ROWS_W)], x_v)
    pltpu.sync_copy(emb_hbm, emb_v)
    pltpu.sync_copy(pos_hbm, pos_v)

    def scale(i, carry):
        x_v[pl.ds(i * L, L)] = x_v[pl.ds(i * L, L)] * D
        return carry

    lax.fori_loop(0, ROWS_W // L, scale, 0)

    posv = [pos_v[pl.ds(L * j, L)] for j in range(P * D // L)]
    lanes = lax.iota(jnp.int32, L)
    rpatb = []
    cpatb = []
    for m in range(3):
        q = lanes + 16 * m
        rpatb.append(lax.div(q, D))
        cpatb.append(lax.rem(q, D))

    def chunk(c, carry):
        row0 = c * CHUNK_ROWS

        def sg(k, carry2):
            r0 = row0 + k * SG_ROWS
            e0 = k * (SG_ROWS * D)
            for t in range(3):
                xs = x_v[pl.ds(r0 + t * L, L)]
                for j2 in range(SG_ROWS * D // L // 3):
                    m = j2 % 3
                    eidx = _dyn_gather(xs, rpatb[m] + (2 * (j2 // 3))) + cpatb[m]
                    val = plsc.load_gather(emb_v, [eidx]) + posv[(6 * t + j2) % 18]
                    buf_v[pl.ds(e0 + (t * 24 + j2) * L, L)] = val
            return carry2

        lax.fori_loop(0, SG_PER_CHUNK, sg, 0)
        pltpu.sync_copy(buf_v,
                        out_hbm.at[pl.ds((base + row0) * D, CHUNK_ELEMS)])
        return carry

    lax.fori_loop(0, NCHUNKS, chunk, 0)


def kernel(x, emb_table, pos_table):
    xf = x.reshape(-1).astype(jnp.int32)
    out = _lookup(xf, emb_table.reshape(-1), pos_table.reshape(-1))
    return out.reshape(B, P, D)


# speedup vs baseline: 9.7802x; 1.7812x over previous
import functools

import jax
import jax.numpy as jnp
from jax import lax
from jax.experimental import pallas as pl
from jax.experimental.pallas import tpu as pltpu
from jax.experimental.pallas import tpu_sc as plsc

B, P, V, D = 16384, 12, 28, 24
N = B * P
NC, NS, L = 2, 16, 16
NW = NC * NS
BT = 128
NBT = B // BT
BT_W = NBT // NW
ROWS_W = BT_W * BT * P
CT = D // 8
SLAB = CT * BT_W * 8 * BT
PCT_BLK = BT_W * 8 * BT
P_STRIDE = CT * NBT * 8 * BT // NBT * NBT
CT_STRIDE = NBT * 8 * BT


_mesh = plsc.VectorSubcoreMesh(core_axis_name="c", subcore_axis_name="s")


@functools.partial(
    pl.kernel,
    mesh=_mesh,
    out_type=jax.ShapeDtypeStruct((N * D,), jnp.float32),
    compiler_params=pltpu.CompilerParams(needs_layout_passes=False),
    scratch_types=[
        pltpu.VMEM((ROWS_W,), jnp.int32),
        pltpu.VMEM((V * D,), jnp.float32),
        pltpu.VMEM((P * D,), jnp.float32),
        pltpu.VMEM((P * V * D,), jnp.float32),
        pltpu.VMEM((2 * SLAB,), jnp.float32),
        pltpu.SemaphoreType.DMA,
        pltpu.SemaphoreType.DMA,
    ],
)
def _lookup(x_hbm, emb_hbm, pos_hbm, out_hbm,
            x_v, emb_v, pos_v, comb_v, buf_v, sem0, sem1):
    wid = lax.axis_index("s") * NC + lax.axis_index("c")
    pltpu.sync_copy(x_hbm.at[pl.ds(wid * ROWS_W, ROWS_W)], x_v)
    pltpu.sync_copy(emb_hbm, emb_v)
    pltpu.sync_copy(pos_hbm, pos_v)

    lanes = lax.iota(jnp.int32, L)
    cpat = [lax.rem(lanes + L * m, D) for m in range(3)]
    xpat = lanes * P
    sems = (sem0, sem1)

    def cb(p, carry):
        s = p * D
        for k in range(V * D // L):
            pv = plsc.load_gather(pos_v, [cpat[k % 3] + s])
            comb_v[pl.ds(p * (V * D) + k * L, L)] = (
                emb_v[pl.ds(k * L, L)] + pv)
        return carry

    lax.fori_loop(0, P, cb, 0)

    out_base = wid * PCT_BLK

    def slab_copies(p, h):
        return [
            pltpu.make_async_copy(
                buf_v.at[pl.ds(h * SLAB + ct * PCT_BLK, PCT_BLK)],
                out_hbm.at[pl.ds(p * P_STRIDE + ct * CT_STRIDE + out_base,
                                 PCT_BLK)],
                sems[h],
            )
            for ct in range(CT)
        ]

    def fill(p, h):
        pb = p * (V * D)
        for bt in range(BT_W):
            for bv in range(BT // L):
                xg = plsc.load_gather(
                    x_v, [xpat + (bt * BT * P + bv * L * P + p)])
                xg24 = xg * D
                for ct in range(CT):
                    for ci in range(8):
                        val = plsc.load_gather(
                            comb_v, [xg24 + (pb + ct * 8 + ci)])
                        buf_v[pl.ds(h * SLAB + ct * PCT_BLK + bt * (8 * BT)
                                    + ci * BT + bv * L, L)] = val

    def outer(p2, carry):
        for h in range(2):
            p = p2 * 2 + h

            @pl.when(p2 > 0)
            def _():
                for cp in slab_copies(p - 2, h):
                    cp.wait()

            fill(p, h)
            for cp in slab_copies(p, h):
                cp.start()
        return carry

    lax.fori_loop(0, P // 2, outer, 0)
    for h in range(2):
        for cp in slab_copies(P - 2 + h, h):
            cp.wait()


def kernel(x, emb_table, pos_table):
    xf = x.reshape(-1).astype(jnp.int32)
    out = _lookup(xf, emb_table.reshape(-1), pos_table.reshape(-1))
    return (out.reshape(P, CT, NBT, 8, BT)
            .transpose(2, 4, 0, 1, 3)
            .reshape(B, P, D))


# speedup vs baseline: 24.3021x; 2.4848x over previous
import functools

import jax
import jax.numpy as jnp
from jax import lax
from jax.experimental import pallas as pl
from jax.experimental.pallas import tpu as pltpu
from jax.experimental.pallas import tpu_sc as plsc

B, P, V, D = 16384, 12, 28, 24
N = B * P
NC, NS, L = 2, 16, 16
NW = NC * NS
BT = 128
NBT = B // BT
BT_W = NBT // NW
ROWS_W = BT_W * BT * P
CT = D // 8
SLAB = CT * BT_W * 8 * BT
PCT_BLK = BT_W * 8 * BT
P_STRIDE = CT * 8 * B
CT_STRIDE = NBT * 8 * BT
VB = 32


_mesh = plsc.VectorSubcoreMesh(core_axis_name="c", subcore_axis_name="s")


@functools.partial(
    pl.kernel,
    mesh=_mesh,
    out_type=jax.ShapeDtypeStruct((N * D,), jnp.float32),
    compiler_params=pltpu.CompilerParams(needs_layout_passes=False),
    scratch_types=[
        pltpu.VMEM((ROWS_W,), jnp.int32),
        pltpu.VMEM((768,), jnp.float32),
        pltpu.VMEM((P * D,), jnp.float32),
        pltpu.VMEM((P * D * VB,), jnp.float32),
        pltpu.VMEM((2 * SLAB,), jnp.float32),
        pltpu.SemaphoreType.DMA,
        pltpu.SemaphoreType.DMA,
    ],
)
def _lookup(x_hbm, emb_hbm, pos_hbm, out_hbm,
            x_v, emb_v, pos_v, comb_v, buf_v, sem0, sem1):
    wid = lax.axis_index("s") * NC + lax.axis_index("c")
    pltpu.sync_copy(x_hbm.at[pl.ds(wid * ROWS_W, ROWS_W)], x_v)
    pltpu.sync_copy(emb_hbm, emb_v.at[pl.ds(0, V * D)])
    pltpu.sync_copy(pos_hbm, pos_v)

    lanes = lax.iota(jnp.int32, L)
    iota24 = lanes * D
    xpat = lanes * P
    zeros = lanes * 0
    sems = (sem0, sem1)

    @plsc.parallel_loop(0, P * D, unroll=4)
    def _(pc):
        c = lax.rem(pc, D)
        g1 = plsc.load_gather(emb_v, [iota24 + c])
        g2 = plsc.load_gather(emb_v, [iota24 + (c + L * D)])
        pv = plsc.load_gather(pos_v, [zeros + pc])
        comb_v[pl.ds(pc * VB, L)] = g1 + pv
        comb_v[pl.ds(pc * VB + L, L)] = g2 + pv

    out_base = wid * PCT_BLK

    def slab_copies(p, h):
        return [
            pltpu.make_async_copy(
                buf_v.at[pl.ds(h * SLAB + ct * PCT_BLK, PCT_BLK)],
                out_hbm.at[pl.ds(p * P_STRIDE + ct * CT_STRIDE + out_base,
                                 PCT_BLK)],
                sems[h],
            )
            for ct in range(CT)
        ]

    def fill(p, h):
        pb = p * (D * VB)

        @plsc.parallel_loop(0, BT_W * (BT // L), unroll=4)
        def _(u):
            bt = lax.shift_right_logical(u, 3)
            bv = lax.bitwise_and(u, 7)
            xg = plsc.load_gather(
                x_v, [xpat + (bt * (BT * P) + bv * (L * P) + p)])
            o0 = h * SLAB + bt * 1024 + bv * L
            for ct in range(CT):
                for ci in range(8):
                    blk = comb_v.at[pl.ds(pb + (ct * 8 + ci) * VB, VB)]
                    val = plsc.load_gather(blk, [xg])
                    buf_v[pl.ds(o0 + ct * PCT_BLK + ci * BT, L)] = val

    def outer(p2, carry):
        for h in range(2):
            p = p2 * 2 + h

            @pl.when(p2 > 0)
            def _():
                for cp in slab_copies(p - 2, h):
                    cp.wait()

            fill(p, h)
            for cp in slab_copies(p, h):
                cp.start()
        return carry

    lax.fori_loop(0, P // 2, outer, 0)
    for h in range(2):
        for cp in slab_copies(P - 2 + h, h):
            cp.wait()


def kernel(x, emb_table, pos_table):
    xf = x.reshape(-1).astype(jnp.int32)
    out = _lookup(xf, emb_table.reshape(-1), pos_table.reshape(-1))
    return (out.reshape(P, CT, NBT, 8, BT)
            .transpose(2, 4, 0, 1, 3)
            .reshape(B, P, D))


# speedup vs baseline: 35.7476x; 1.4710x over previous
import functools

import jax
import jax.numpy as jnp
from jax import lax
from jax.experimental import pallas as pl
from jax.experimental.pallas import tpu as pltpu
from jax.experimental.pallas import tpu_sc as plsc

B, P, V, D = 16384, 12, 28, 24
N = B * P
NC, NS, L = 2, 16, 16
NW = NC * NS
BT = 128
NBT = B // BT
BT_W = NBT // NW
ROWS_W = BT_W * BT * P
CT = D // 8
SLAB = CT * BT_W * 8 * BT
PCT_BLK = BT_W * 8 * BT
P_STRIDE = CT * 8 * B
CT_STRIDE = NBT * 8 * BT
VB = 32


_mesh = plsc.VectorSubcoreMesh(core_axis_name="c", subcore_axis_name="s")


@functools.partial(
    pl.kernel,
    mesh=_mesh,
    out_type=jax.ShapeDtypeStruct((N * D,), jnp.float32),
    compiler_params=pltpu.CompilerParams(needs_layout_passes=False),
    scratch_types=[
        pltpu.VMEM((ROWS_W,), jnp.int32),
        pltpu.VMEM((768,), jnp.float32),
        pltpu.VMEM((P * D,), jnp.float32),
        pltpu.VMEM((P * D * VB,), jnp.float32),
        pltpu.VMEM((2 * SLAB,), jnp.float32),
        pltpu.SemaphoreType.DMA,
        pltpu.SemaphoreType.DMA,
        pltpu.SemaphoreType.DMA,
    ],
)
def _lookup(x_hbm, emb_hbm, pos_hbm, out_hbm,
            x_v, emb_v, pos_v, comb_v, buf_v, sem0, sem1, semx):
    wid = lax.axis_index("s") * NC + lax.axis_index("c")
    col0 = wid * (BT_W * BT)
    xcps = [
        pltpu.make_async_copy(
            x_hbm.at[p, pl.ds(col0, BT_W * BT)],
            x_v.at[pl.ds(p * (BT_W * BT), BT_W * BT)],
            semx,
        )
        for p in range(P)
    ]
    for cp in xcps:
        cp.start()
    pltpu.sync_copy(emb_hbm, emb_v.at[pl.ds(0, V * D)])
    pltpu.sync_copy(pos_hbm, pos_v)

    lanes = lax.iota(jnp.int32, L)
    iota24 = lanes * D
    zeros = lanes * 0
    sems = (sem0, sem1)

    @plsc.parallel_loop(0, P * D, unroll=4)
    def _(pc):
        c = lax.rem(pc, D)
        g1 = plsc.load_gather(emb_v, [iota24 + c])
        g2 = plsc.load_gather(emb_v, [iota24 + (c + L * D)])
        pv = plsc.load_gather(pos_v, [zeros + pc])
        comb_v[pl.ds(pc * VB, L)] = g1 + pv
        comb_v[pl.ds(pc * VB + L, L)] = g2 + pv

    for cp in xcps:
        cp.wait()

    out_base = wid * PCT_BLK

    def slab_copies(p, h):
        return [
            pltpu.make_async_copy(
                buf_v.at[pl.ds(h * SLAB + ct * PCT_BLK, PCT_BLK)],
                out_hbm.at[pl.ds(p * P_STRIDE + ct * CT_STRIDE + out_base,
                                 PCT_BLK)],
                sems[h],
            )
            for ct in range(CT)
        ]

    def fill(p, h):
        pb = p * (D * VB)

        @plsc.parallel_loop(0, BT_W * (BT // L), unroll=4)
        def _(u):
            bt = lax.shift_right_logical(u, 3)
            bv = lax.bitwise_and(u, 7)
            xg = x_v[pl.ds(p * (BT_W * BT) + u * L, L)]
            o0 = h * SLAB + bt * 1024 + bv * L
            for ct in range(CT):
                for ci in range(8):
                    blk = comb_v.at[pl.ds(pb + (ct * 8 + ci) * VB, VB)]
                    val = plsc.load_gather(blk, [xg])
                    buf_v[pl.ds(o0 + ct * PCT_BLK + ci * BT, L)] = val

    def outer(p2, carry):
        for h in range(2):
            p = p2 * 2 + h

            @pl.when(p2 > 0)
            def _():
                for cp in slab_copies(p - 2, h):
                    cp.wait()

            fill(p, h)
            for cp in slab_copies(p, h):
                cp.start()
        return carry

    lax.fori_loop(0, P // 2, outer, 0)
    for h in range(2):
        for cp in slab_copies(P - 2 + h, h):
            cp.wait()


def kernel(x, emb_table, pos_table):
    xf = x.T.astype(jnp.int32)
    out = _lookup(xf, emb_table.reshape(-1), pos_table.reshape(-1))
    return (out.reshape(P, CT, NBT, 8, BT)
            .transpose(2, 4, 0, 1, 3)
            .reshape(B, P, D))
